# baseline (speedup 1.0000x reference)
import functools

import jax
import jax.numpy as jnp
from jax import lax
from jax.experimental import pallas as pl
from jax.experimental.pallas import tpu as pltpu
from jax.experimental.pallas import tpu_sc as plsc

NUM_CLASSES = 1000
HIDDEN = 1152
BATCH = 16384
UNCOND_ID = NUM_CLASSES

NC = 2
NS = 16
L = 16
NW = NC * NS
B_PER_W = BATCH // NW
CHUNK = 16
NCHUNK = B_PER_W // CHUNK
NBUF = 4
TROWS = NUM_CLASSES + 1


def _make_kernel():
    mesh = plsc.VectorSubcoreMesh(core_axis_name="c", subcore_axis_name="s")

    @functools.partial(
        pl.kernel,
        mesh=mesh,
        out_type=jax.ShapeDtypeStruct((BATCH, HIDDEN), jnp.float32),
        scratch_types=(
            [pltpu.VMEM((B_PER_W,), jnp.int32)] * 2
            + [pltpu.VMEM((NCHUNK, CHUNK), jnp.int32)]
            + [pltpu.VMEM((CHUNK, HIDDEN), jnp.float32)] * NBUF
            + [pltpu.SemaphoreType.DMA] * (2 * NBUF)
        ),
    )
    def emb_kernel(labels_hbm, drop_hbm, table_hbm, out_hbm,
                   lab_v, drop_v, idx_v, *bufs_sems):
        bufs = bufs_sems[:NBUF]
        gsem = bufs_sems[NBUF:2 * NBUF]
        ssem = bufs_sems[2 * NBUF:]
        sid = lax.axis_index("s")
        wid = sid * NC + lax.axis_index("c")
        base = wid * B_PER_W

        pltpu.sync_copy(labels_hbm.at[pl.ds(base, B_PER_W)], lab_v)
        pltpu.sync_copy(drop_hbm.at[pl.ds(base, B_PER_W)], drop_v)

        for i in range(B_PER_W // L):
            lab = lab_v[pl.ds(i * L, L)]
            dr = drop_v[pl.ds(i * L, L)]
            idx_v[i // (CHUNK // L), pl.ds((i % (CHUNK // L)) * L, L)] = (
                jnp.where(dr != 0, jnp.full((L,), UNCOND_ID, jnp.int32), lab))

        def gath(c, slot):
            return pltpu.make_async_copy(
                table_hbm.at[idx_v.at[c]], bufs[slot], gsem[slot])

        def scat(c, slot):
            return pltpu.make_async_copy(
                bufs[slot], out_hbm.at[pl.ds(base + c * CHUNK, CHUNK)],
                ssem[slot])

        gath(0, 0).start()
        for c in range(NCHUNK):
            slot = c % NBUF
            nxt = c + 1
            if nxt < NCHUNK:
                ns = nxt % NBUF
                if nxt >= NBUF:
                    scat(nxt - NBUF, ns).wait()
                gath(nxt, ns).start()
            gath(c, slot).wait()
            scat(c, slot).start()
        for c in range(max(0, NCHUNK - NBUF), NCHUNK):
            scat(c, c % NBUF).wait()

    return emb_kernel


_emb_kernel = _make_kernel()


def kernel(labels, train, force_drop_ids, table):
    del train
    return _emb_kernel(labels.astype(jnp.int32),
                       force_drop_ids.astype(jnp.int32),
                       table)
